# baseline (speedup 1.0000x reference)
import functools

import jax
import jax.numpy as jnp
from jax import lax
from jax.experimental import pallas as pl
from jax.experimental.pallas import tpu as pltpu
from jax.experimental.pallas import tpu_sc as plsc

N_NODES = 10000
N_EDGES = 320000
P_CLUSTERS = 512
H_DIM = 64

NC = 2
NS = 16
NW = NC * NS

BB = 1280
G = 128
CAPL = 2560
CAPR = CAPL + G
SEG = 16 * CAPR
TR = 4096


def _wid():
    return lax.axis_index("s") * NC + lax.axis_index("c")


def _make_bucket_kernel(e_pad, npt):
    nblk = e_pad // BB
    nrow = BB // G
    mesh = plsc.VectorSubcoreMesh(core_axis_name="c", subcore_axis_name="s")

    @functools.partial(
        pl.kernel,
        mesh=mesh,
        out_type=(
            jax.ShapeDtypeStruct((NW * SEG + NW * TR,), jnp.int32),
            jax.ShapeDtypeStruct((NW * 16,), jnp.int32),
        ),
        scratch_types=[
            pltpu.VMEM((BB,), jnp.int32),
            pltpu.VMEM((BB,), jnp.int32),
            pltpu.VMEM((16, G), jnp.int32),
            pltpu.VMEM((16, G), jnp.int32),
            pltpu.VMEM((16,), jnp.int32),
            pltpu.SemaphoreType.DMA,
        ],
    )
    def bucket(src_hbm, dst_hbm, opk, ocnt, sv, dv, pstage,
               istage, c16, sem):
        wid = _wid()
        lo = wid * npt
        lane = jnp.arange(16, dtype=jnp.int32)
        base_v = lane * CAPR + wid * SEG
        tbase = NW * SEG + wid * TR
        capl = jnp.full((16,), CAPL, dtype=jnp.int32)

        def flush(rows):
            hs = []
            for j in range(rows):
                hs.append(pltpu.async_copy(
                    pstage.at[j], opk.at[istage.at[j]], sem))
            for h in hs:
                h.wait()

        def blk(b, cnt_v):
            pltpu.sync_copy(src_hbm.at[pl.ds(pl.multiple_of(b * BB, 8), BB)], sv)
            pltpu.sync_copy(dst_hbm.at[pl.ds(pl.multiple_of(b * BB, 8), BB)], dv)

            def row(r, cv):
                for q in range(G // 16):
                    o = (r * (G // 16) + q) * 16
                    s16 = sv[pl.ds(o, 16)]
                    d16 = dv[pl.ds(o, 16)]
                    dl = d16 - lo
                    m = plsc.bitcast(dl, jnp.uint32) < jnp.uint32(npt)
                    toff = (o * 2) % TR
                    trash_v = tbase + toff + lane
                    pos = jnp.where(m, base_v + cv, trash_v)
                    pos = jnp.where(cv < capl, pos, trash_v)
                    pstage[r, pl.ds(q * 16, 16)] = s16 * 512 + dl
                    istage[r, pl.ds(q * 16, 16)] = pos
                    mi = jnp.where(m, jnp.int32(1), jnp.int32(0))
                    cv = jnp.minimum(cv + mi, capl)
                return cv

            cnt_v = lax.fori_loop(0, nrow, row, cnt_v)
            flush(nrow)
            return cnt_v

        cnt_v = lax.fori_loop(0, nblk, blk, jnp.zeros((16,), jnp.int32))

        dumm = jnp.full((16,), npt, dtype=jnp.int32)
        for j in range(16):
            for q in range(G // 16):
                k = j * (G // 16) + q
                pstage[j, pl.ds(q * 16, 16)] = dumm
                istage[j, pl.ds(q * 16, 16)] = base_v + cnt_v + k
        flush(16)

        c16[pl.ds(0, 16)] = cnt_v
        pltpu.sync_copy(c16, ocnt.at[pl.ds(pl.multiple_of(wid * 16, 8), 16)])

    return bucket


def _make_scatter_max_kernel(n_rows, npt):
    mesh = plsc.VectorSubcoreMesh(core_axis_name="c", subcore_axis_name="s")
    neg = jnp.float32(-jnp.inf)
    seg = 16 * CAPR

    @functools.partial(
        pl.kernel,
        mesh=mesh,
        out_type=jax.ShapeDtypeStruct((NW * npt, 128), jnp.float32),
        scratch_types=[
            pltpu.VMEM((G,), jnp.int32),
            pltpu.VMEM((G,), jnp.int32),
            pltpu.VMEM((G,), jnp.int32),
            pltpu.VMEM((G, 128), jnp.float32),
            pltpu.VMEM((16,), jnp.int32),
            pltpu.SMEM((16,), jnp.int32),
            pltpu.VMEM((npt + 1, 128), jnp.float32),
            pltpu.SemaphoreType.DMA,
        ],
    )
    def scatter_max(x_hbm, opk, ocnt, out_hbm, gp, gs, gd, rows, c16, sc, agg, sem):
        wid = _wid()
        pltpu.sync_copy(ocnt.at[pl.ds(pl.multiple_of(wid * 16, 8), 16)], c16)
        cv = c16[pl.ds(0, 16)]
        for l in range(16):
            sc[l] = cv[l]

        def initrow(r, _):
            for j in range(8):
                agg[r, pl.ds(j * 16, 16)] = jnp.full((16,), neg, dtype=jnp.float32)
            return 0

        lax.fori_loop(0, npt + 1, initrow, 0)

        hbase = wid * seg

        def lane_body(l, _):
            cl = sc[l]
            ngl = lax.div(cl + (G - 1), G)
            lbase = hbase + l * CAPR

            def gran(g, _):
                off = pl.multiple_of(lbase + g * G, 8)
                pltpu.sync_copy(opk.at[pl.ds(off, G)], gp)
                for q in range(G // 16):
                    s = pl.ds(q * 16, 16)
                    pk = gp[s]
                    sv16 = lax.shift_right_logical(pk, 9)
                    gs[s] = sv16
                    gd[s] = pk - sv16 * 512
                pltpu.async_copy(x_hbm.at[gs], rows, sem).wait()

                def grp(gi, _):
                    dgrp = gd[pl.ds(gi * 16, 16)]
                    for q in range(16):
                        dl = dgrp[q]
                        r = gi * 16 + q
                        for j in range(8):
                            s = pl.ds(j * 16, 16)
                            agg[dl, s] = jnp.maximum(agg[dl, s], rows[r, s])
                    return 0

                lax.fori_loop(0, G // 16, grp, 0)
                return 0

            lax.fori_loop(0, ngl, gran, 0)
            return 0

        lax.fori_loop(0, 16, lane_body, 0)

        def wb(r, _):
            for j in range(8):
                s = pl.ds(j * 16, 16)
                v = agg[r, s]
                agg[r, s] = jnp.where(v == neg, jnp.float32(0.0), v)
            return 0

        lax.fori_loop(0, npt, wb, 0)
        pltpu.sync_copy(agg.at[pl.ds(0, npt)], out_hbm.at[pl.ds(pl.multiple_of(wid * npt, 8), npt)])

    return scatter_max


def _mlp_tc(x, w1, b1, g, beta, w2, b2):
    n, d = x.shape
    bn = 1000
    p1 = jnp.broadcast_to(jnp.stack([b1, g, beta], 0), (3, H_DIM))
    p2 = jnp.broadcast_to(b2[None, :], (8, d))

    def body(x_ref, w1_ref, p1_ref, w2_ref, p2_ref, o_ref):
        xb = x_ref[...]
        h = jnp.dot(xb, w1_ref[...], preferred_element_type=jnp.float32)
        h = h + p1_ref[0:1, :]
        mu = jnp.mean(h, axis=-1, keepdims=True)
        var = jnp.mean((h - mu) ** 2, axis=-1, keepdims=True)
        h = (h - mu) * lax.rsqrt(var + 1e-5) * p1_ref[1:2, :] + p1_ref[2:3, :]
        h = jnp.maximum(h, 0.0)
        o = jnp.dot(h, w2_ref[...], preferred_element_type=jnp.float32)
        o_ref[...] = o + p2_ref[0:1, :]

    return pl.pallas_call(
        body,
        grid=(n // bn,),
        in_specs=[
            pl.BlockSpec((bn, d), lambda i: (i, 0)),
            pl.BlockSpec((d, H_DIM), lambda i: (0, 0)),
            pl.BlockSpec((3, H_DIM), lambda i: (0, 0)),
            pl.BlockSpec((H_DIM, d), lambda i: (0, 0)),
            pl.BlockSpec((8, d), lambda i: (0, 0)),
        ],
        out_specs=pl.BlockSpec((bn, d), lambda i: (i, 0)),
        out_shape=jax.ShapeDtypeStruct((n, d), jnp.float32),
    )(x, w1, p1, w2, p2)


def _normalize_tc(pooled):
    def body(p_ref, o_ref):
        p = p_ref[...]
        nrm = jnp.sqrt(jnp.sum(p * p, axis=0, keepdims=True))
        nrm = jnp.where(nrm == 0.0, jnp.ones_like(nrm), nrm)
        o_ref[...] = p / nrm

    return pl.pallas_call(
        body,
        out_shape=jax.ShapeDtypeStruct(pooled.shape, jnp.float32),
    )(pooled)


def _segment_max(x_t, lists, npt, n_out):
    opk, ocnt = lists
    n, d = x_t.shape
    kern = _make_scatter_max_kernel(n, npt)
    cols = []
    for c in range(d // 128):
        xc = x_t[:, c * 128:(c + 1) * 128]
        cols.append(kern(xc, opk, ocnt)[:n_out])
    return jnp.concatenate(cols, axis=1)


def kernel(x, edge_index, cluster,
           w1_0, b1_0, g_0, beta_0, w2_0, b2_0,
           w1_1, b1_1, g_1, beta_1, w2_1, b2_1,
           w1_2, b1_2, g_2, beta_2, w2_2, b2_2):
    src = edge_index[0]
    dst = edge_index[1]

    npt_e = 320
    bucket_e = _make_bucket_kernel(N_EDGES, npt_e)
    lists_e = bucket_e(src, dst)

    e_pool = 10240
    npt_p = P_CLUSTERS // NW
    ids = jnp.arange(e_pool, dtype=jnp.int32) % N_NODES
    cl_pad = jnp.concatenate(
        [cluster.astype(jnp.int32),
         jnp.full((e_pool - N_NODES,), -1, dtype=jnp.int32)])
    bucket_p = _make_bucket_kernel(e_pool, npt_p)
    lists_p = bucket_p(ids, cl_pad)

    params = [
        (w1_0, b1_0, g_0, beta_0, w2_0, b2_0),
        (w1_1, b1_1, g_1, beta_1, w2_1, b2_1),
        (w1_2, b1_2, g_2, beta_2, w2_2, b2_2),
    ]
    cur = x
    for layer in range(3):
        x_t = _mlp_tc(cur, *params[layer])
        agg = _segment_max(x_t, lists_e, npt_e, N_NODES)
        cur = jnp.concatenate([x_t, agg], axis=1)

    pooled = _segment_max(cur, lists_p, npt_p, P_CLUSTERS)
    return _normalize_tc(pooled)


# speedup vs baseline: 1.1116x; 1.1116x over previous
import functools

import jax
import jax.numpy as jnp
from jax import lax
from jax.experimental import pallas as pl
from jax.experimental.pallas import tpu as pltpu
from jax.experimental.pallas import tpu_sc as plsc

N_NODES = 10000
N_EDGES = 320000
P_CLUSTERS = 512
H_DIM = 64

NC = 2
NS = 16
NW = NC * NS

G = 128
CAPL = 2560
CAPR = CAPL + G
SEG = 16 * CAPR


def _wid():
    return lax.axis_index("s") * NC + lax.axis_index("c")


def _make_bucket_kernel(e_pad, npt, bb):
    nblk = e_pad // bb
    nrow = bb // G
    srow = max(nrow, 16)
    tr = 2 * bb
    mesh = plsc.VectorSubcoreMesh(core_axis_name="c", subcore_axis_name="s")

    @functools.partial(
        pl.kernel,
        mesh=mesh,
        out_type=(
            jax.ShapeDtypeStruct((NW * SEG + NW * tr,), jnp.int32),
            jax.ShapeDtypeStruct((NW * 16,), jnp.int32),
        ),
        scratch_types=[
            pltpu.VMEM((bb,), jnp.int32),
            pltpu.VMEM((bb,), jnp.int32),
            pltpu.VMEM((srow, G), jnp.int32),
            pltpu.VMEM((srow, G), jnp.int32),
            pltpu.VMEM((16,), jnp.int32),
            pltpu.SemaphoreType.DMA,
        ],
    )
    def bucket(src_hbm, dst_hbm, opk, ocnt, sv, dv, pstage,
               istage, c16, sem):
        wid = _wid()
        lo = wid * npt
        lane = jnp.arange(16, dtype=jnp.int32)
        base_v = lane * CAPR + wid * SEG
        tbase = NW * SEG + wid * tr
        capl = jnp.full((16,), CAPL, dtype=jnp.int32)

        def flush(rows):
            hs = []
            for j in range(rows):
                hs.append(pltpu.async_copy(
                    pstage.at[j], opk.at[istage.at[j]], sem))
            for h in hs:
                h.wait()

        def blk(b, cnt_v):
            pltpu.sync_copy(src_hbm.at[pl.ds(pl.multiple_of(b * bb, 8), bb)], sv)
            pltpu.sync_copy(dst_hbm.at[pl.ds(pl.multiple_of(b * bb, 8), bb)], dv)

            def row(r, cv):
                for q in range(G // 16):
                    o = (r * (G // 16) + q) * 16
                    s16 = sv[pl.ds(o, 16)]
                    d16 = dv[pl.ds(o, 16)]
                    dl = d16 - lo
                    m = plsc.bitcast(dl, jnp.uint32) < jnp.uint32(npt)
                    toff = (o * 2) % tr
                    trash_v = tbase + toff + lane
                    pos = jnp.where(m, base_v + cv, trash_v)
                    pos = jnp.where(cv < capl, pos, trash_v)
                    pstage[r, pl.ds(q * 16, 16)] = s16 * 512 + dl
                    istage[r, pl.ds(q * 16, 16)] = pos
                    mi = jnp.where(m, jnp.int32(1), jnp.int32(0))
                    cv = jnp.minimum(cv + mi, capl)
                return cv

            cnt_v = lax.fori_loop(0, nrow, row, cnt_v)
            flush(nrow)
            return cnt_v

        cnt_v = lax.fori_loop(0, nblk, blk, jnp.zeros((16,), jnp.int32))

        dumm = jnp.full((16,), npt, dtype=jnp.int32)
        for j in range(16):
            for q in range(G // 16):
                k = j * (G // 16) + q
                pstage[j, pl.ds(q * 16, 16)] = dumm
                istage[j, pl.ds(q * 16, 16)] = base_v + cnt_v + k
        flush(16)

        c16[pl.ds(0, 16)] = cnt_v
        pltpu.sync_copy(c16, ocnt.at[pl.ds(pl.multiple_of(wid * 16, 8), 16)])

    return bucket


def _make_scatter_max_kernel(n_rows, npt):
    mesh = plsc.VectorSubcoreMesh(core_axis_name="c", subcore_axis_name="s")
    neg = jnp.float32(-jnp.inf)
    seg = 16 * CAPR

    @functools.partial(
        pl.kernel,
        mesh=mesh,
        out_type=jax.ShapeDtypeStruct((NW * npt, 128), jnp.float32),
        scratch_types=[
            pltpu.VMEM((G,), jnp.int32),
            pltpu.VMEM((G,), jnp.int32),
            pltpu.VMEM((G,), jnp.int32),
            pltpu.VMEM((G, 128), jnp.float32),
            pltpu.VMEM((16,), jnp.int32),
            pltpu.SMEM((16,), jnp.int32),
            pltpu.VMEM((npt + 1, 128), jnp.float32),
            pltpu.SemaphoreType.DMA,
        ],
    )
    def scatter_max(x_hbm, opk, ocnt, out_hbm, gp, gs, gd, rows, c16, sc, agg, sem):
        wid = _wid()
        pltpu.sync_copy(ocnt.at[pl.ds(pl.multiple_of(wid * 16, 8), 16)], c16)
        cv = c16[pl.ds(0, 16)]
        for l in range(16):
            sc[l] = cv[l]

        def initrow(r, _):
            for j in range(8):
                agg[r, pl.ds(j * 16, 16)] = jnp.full((16,), neg, dtype=jnp.float32)
            return 0

        lax.fori_loop(0, npt + 1, initrow, 0)

        hbase = wid * seg

        def lane_body(l, _):
            cl = sc[l]
            ngl = lax.div(cl + (G - 1), G)
            lbase = hbase + l * CAPR

            def gran(g, _):
                off = pl.multiple_of(lbase + g * G, 8)
                pltpu.sync_copy(opk.at[pl.ds(off, G)], gp)
                for q in range(G // 16):
                    s = pl.ds(q * 16, 16)
                    pk = gp[s]
                    sv16 = lax.shift_right_logical(pk, 9)
                    gs[s] = sv16
                    gd[s] = pk - sv16 * 512
                pltpu.async_copy(x_hbm.at[gs], rows, sem).wait()

                def grp(gi, _):
                    dgrp = gd[pl.ds(gi * 16, 16)]
                    for q in range(16):
                        dl = dgrp[q]
                        r = gi * 16 + q
                        for j in range(8):
                            s = pl.ds(j * 16, 16)
                            agg[dl, s] = jnp.maximum(agg[dl, s], rows[r, s])
                    return 0

                lax.fori_loop(0, G // 16, grp, 0)
                return 0

            lax.fori_loop(0, ngl, gran, 0)
            return 0

        lax.fori_loop(0, 16, lane_body, 0)

        def wb(r, _):
            for j in range(8):
                s = pl.ds(j * 16, 16)
                v = agg[r, s]
                agg[r, s] = jnp.where(v == neg, jnp.float32(0.0), v)
            return 0

        lax.fori_loop(0, npt, wb, 0)
        pltpu.sync_copy(agg.at[pl.ds(0, npt)], out_hbm.at[pl.ds(pl.multiple_of(wid * npt, 8), npt)])

    return scatter_max


def _mlp_tc(x, w1, b1, g, beta, w2, b2):
    n, d = x.shape
    bn = 1000
    p1 = jnp.broadcast_to(jnp.stack([b1, g, beta], 0), (3, H_DIM))
    p2 = jnp.broadcast_to(b2[None, :], (8, d))

    def body(x_ref, w1_ref, p1_ref, w2_ref, p2_ref, o_ref):
        xb = x_ref[...]
        h = jnp.dot(xb, w1_ref[...], preferred_element_type=jnp.float32)
        h = h + p1_ref[0:1, :]
        mu = jnp.mean(h, axis=-1, keepdims=True)
        var = jnp.mean((h - mu) ** 2, axis=-1, keepdims=True)
        h = (h - mu) * lax.rsqrt(var + 1e-5) * p1_ref[1:2, :] + p1_ref[2:3, :]
        h = jnp.maximum(h, 0.0)
        o = jnp.dot(h, w2_ref[...], preferred_element_type=jnp.float32)
        o_ref[...] = o + p2_ref[0:1, :]

    return pl.pallas_call(
        body,
        grid=(n // bn,),
        in_specs=[
            pl.BlockSpec((bn, d), lambda i: (i, 0)),
            pl.BlockSpec((d, H_DIM), lambda i: (0, 0)),
            pl.BlockSpec((3, H_DIM), lambda i: (0, 0)),
            pl.BlockSpec((H_DIM, d), lambda i: (0, 0)),
            pl.BlockSpec((8, d), lambda i: (0, 0)),
        ],
        out_specs=pl.BlockSpec((bn, d), lambda i: (i, 0)),
        out_shape=jax.ShapeDtypeStruct((n, d), jnp.float32),
    )(x, w1, p1, w2, p2)


def _normalize_tc(pooled):
    def body(p_ref, o_ref):
        p = p_ref[...]
        nrm = jnp.sqrt(jnp.sum(p * p, axis=0, keepdims=True))
        nrm = jnp.where(nrm == 0.0, jnp.ones_like(nrm), nrm)
        o_ref[...] = p / nrm

    return pl.pallas_call(
        body,
        out_shape=jax.ShapeDtypeStruct(pooled.shape, jnp.float32),
    )(pooled)


def _segment_max(x_t, lists, npt, n_out):
    opk, ocnt = lists
    n, d = x_t.shape
    kern = _make_scatter_max_kernel(n, npt)
    cols = []
    for c in range(d // 128):
        xc = x_t[:, c * 128:(c + 1) * 128]
        cols.append(kern(xc, opk, ocnt)[:n_out])
    return jnp.concatenate(cols, axis=1)


def kernel(x, edge_index, cluster,
           w1_0, b1_0, g_0, beta_0, w2_0, b2_0,
           w1_1, b1_1, g_1, beta_1, w2_1, b2_1,
           w1_2, b1_2, g_2, beta_2, w2_2, b2_2):
    src = edge_index[0]
    dst = edge_index[1]

    npt_e = 320
    bucket_e = _make_bucket_kernel(N_EDGES, npt_e, 6400)
    lists_e = bucket_e(src, dst)

    e_pool = 10240
    npt_p = P_CLUSTERS // NW
    ids = jnp.arange(e_pool, dtype=jnp.int32) % N_NODES
    cl_pad = jnp.concatenate(
        [cluster.astype(jnp.int32),
         jnp.full((e_pool - N_NODES,), -1, dtype=jnp.int32)])
    bucket_p = _make_bucket_kernel(e_pool, npt_p, 2560)
    lists_p = bucket_p(ids, cl_pad)

    params = [
        (w1_0, b1_0, g_0, beta_0, w2_0, b2_0),
        (w1_1, b1_1, g_1, beta_1, w2_1, b2_1),
        (w1_2, b1_2, g_2, beta_2, w2_2, b2_2),
    ]
    cur = x
    for layer in range(3):
        x_t = _mlp_tc(cur, *params[layer])
        agg = _segment_max(x_t, lists_e, npt_e, N_NODES)
        cur = jnp.concatenate([x_t, agg], axis=1)

    pooled = _segment_max(cur, lists_p, npt_p, P_CLUSTERS)
    return _normalize_tc(pooled)
